# baseline (speedup 1.0000x reference)
import functools

import jax
import jax.numpy as jnp
from jax import lax
from jax.experimental import pallas as pl
from jax.experimental.pallas import tpu as pltpu
from jax.experimental.pallas import tpu_sc as plsc

R2 = 0.81
KNB = 32
NPT = 4096
MPT = NPT // 4
NFRAME = 12
CHUNKQ = 128
NCHUNK = MPT // CHUNKQ
LANES = 16
FPS_UNROLL = 4
BQ_UNROLL = 4

_f32 = jnp.float32
_i32 = jnp.int32


def _build_sc_kernel():
    info = plsc.get_sparse_core_info()
    nc = info.num_cores
    mesh = plsc.VectorSubcoreMesh(core_axis_name="c", subcore_axis_name="s")
    out_type = (
        jax.ShapeDtypeStruct((NFRAME, MPT), _f32),
        jax.ShapeDtypeStruct((NFRAME, MPT), _f32),
        jax.ShapeDtypeStruct((NFRAME, MPT), _f32),
        jax.ShapeDtypeStruct((NFRAME, KNB, MPT), _f32),
        jax.ShapeDtypeStruct((NFRAME, KNB, MPT), _f32),
        jax.ShapeDtypeStruct((NFRAME, KNB, MPT), _f32),
    )
    scratch = [
        pltpu.VMEM((NPT,), _f32),
        pltpu.VMEM((NPT,), _f32),
        pltpu.VMEM((NPT,), _f32),
        pltpu.VMEM((NPT,), _f32),
        pltpu.VMEM((MPT,), _f32),
        pltpu.VMEM((MPT,), _f32),
        pltpu.VMEM((MPT,), _f32),
        pltpu.VMEM((KNB, CHUNKQ), _f32),
        pltpu.VMEM((KNB, CHUNKQ), _f32),
        pltpu.VMEM((KNB, CHUNKQ), _f32),
    ]

    @functools.partial(
        pl.kernel, out_type=out_type, mesh=mesh, scratch_types=scratch,
        compiler_params=pltpu.CompilerParams(needs_layout_passes=False))
    def sc_kernel(xs_h, ys_h, zs_h, rfx_h, rfy_h, rfz_h, dpx_h, dpy_h, dpz_h,
                  xs, ys, zs, dist, rfx, rfy, rfz, bx, by, bz):
        wid = lax.axis_index("s") * nc + lax.axis_index("c")

        @pl.when(wid < NFRAME)
        def _frame():
            f = wid
            pltpu.sync_copy(xs_h.at[f], xs)
            pltpu.sync_copy(ys_h.at[f], ys)
            pltpu.sync_copy(zs_h.at[f], zs)

            iota = lax.iota(_i32, LANES)
            lane0 = iota == 0
            big = jnp.full((LANES,), 1e10, _f32)

            @plsc.parallel_loop(0, NPT // LANES, unroll=8)
            def _init(j):
                dist[pl.ds(j * LANES, LANES)] = big

            def fps_step(i, far):
                farv = jnp.full((LANES,), far, _i32)
                cxv = plsc.load_gather(xs, [farv])
                cyv = plsc.load_gather(ys, [farv])
                czv = plsc.load_gather(zs, [farv])
                iv = jnp.full((LANES,), i, _i32)
                plsc.store_scatter(rfx, [iv], cxv, mask=lane0)
                plsc.store_scatter(rfy, [iv], cyv, mask=lane0)
                plsc.store_scatter(rfz, [iv], czv, mask=lane0)

                neg = jnp.full((LANES,), -1.0, _f32)
                zi = jnp.zeros((LANES,), _i32)

                @plsc.parallel_loop(0, NPT // (LANES * FPS_UNROLL), unroll=4,
                                    carry=((neg,) * FPS_UNROLL,
                                           (zi,) * FPS_UNROLL))
                def inner(j, acc):
                    bv, bi = list(acc[0]), list(acc[1])
                    base = j * (LANES * FPS_UNROLL)
                    for u in range(FPS_UNROLL):
                        off = base + u * LANES
                        dx = xs[pl.ds(off, LANES)] - cxv
                        dy = ys[pl.ds(off, LANES)] - cyv
                        dz = zs[pl.ds(off, LANES)] - czv
                        d = dx * dx + dy * dy + dz * dz
                        nd = jnp.minimum(dist[pl.ds(off, LANES)], d)
                        dist[pl.ds(off, LANES)] = nd
                        gt = nd > bv[u]
                        bv[u] = jnp.where(gt, nd, bv[u])
                        bi[u] = jnp.where(gt, iota + off, bi[u])
                    return (tuple(bv), tuple(bi))

                bv, bi = inner

                def comb(va, ia, vb, ib):
                    t = (vb > va) | ((vb == va) & (ib < ia))
                    return jnp.where(t, vb, va), jnp.where(t, ib, ia)

                v01, i01 = comb(bv[0], bi[0], bv[1], bi[1])
                v23, i23 = comb(bv[2], bi[2], bv[3], bi[3])
                v, ivec = comb(v01, i01, v23, i23)
                mx = jnp.max(v)
                cand = jnp.where(v == mx, ivec, jnp.full((LANES,), 1 << 30, _i32))
                return jnp.min(cand)

            lax.fori_loop(0, MPT, fps_step, jnp.int32(0))

            pltpu.sync_copy(rfx, rfx_h.at[f])
            pltpu.sync_copy(rfy, rfy_h.at[f])
            pltpu.sync_copy(rfz, rfz_h.at[f])

            ones = jnp.ones((LANES,), _i32)

            def chunk_body(cidx, carry):
                def q_body(ml, carry2):
                    m = cidx * CHUNKQ + ml
                    mv = jnp.full((LANES,), m, _i32)
                    qxv = plsc.load_gather(rfx, [mv])
                    qyv = plsc.load_gather(rfy, [mv])
                    qzv = plsc.load_gather(rfz, [mv])
                    mlv = jnp.full((LANES,), ml, _i32)

                    def cond(st):
                        j, cnt = st
                        return (cnt < KNB) & (j < NPT // (LANES * BQ_UNROLL))

                    def body(st):
                        j, cnt = st
                        cntv = jnp.full((LANES,), cnt, _i32)
                        base = j * (LANES * BQ_UNROLL)
                        for u in range(BQ_UNROLL):
                            off = base + u * LANES
                            dx = xs[pl.ds(off, LANES)] - qxv
                            dy = ys[pl.ds(off, LANES)] - qyv
                            dz = zs[pl.ds(off, LANES)] - qzv
                            d2 = dx * dx + dy * dy + dz * dz
                            msk = d2 < R2
                            pos = cntv + plsc.cumsum(ones, mask=msk) - 1
                            smask = msk & (pos < KNB)
                            plsc.store_scatter(bx, [pos, mlv], dx, mask=smask)
                            plsc.store_scatter(by, [pos, mlv], dy, mask=smask)
                            plsc.store_scatter(bz, [pos, mlv], dz, mask=smask)
                            cntv = cntv + plsc.all_reduce_population_count(msk)
                        return (j + 1, cntv[0])

                    _, cnt = lax.while_loop(cond, body,
                                            (jnp.int32(0), jnp.int32(0)))

                    cntv = jnp.full((LANES,), cnt, _i32)
                    zi = jnp.zeros((LANES,), _i32)
                    d0x = plsc.load_gather(bx, [zi, mlv])
                    d0y = plsc.load_gather(by, [zi, mlv])
                    d0z = plsc.load_gather(bz, [zi, mlv])
                    for kk in range(KNB // LANES):
                        kvec = iota + kk * LANES
                        fm = kvec >= cntv
                        plsc.store_scatter(bx, [kvec, mlv], d0x, mask=fm)
                        plsc.store_scatter(by, [kvec, mlv], d0y, mask=fm)
                        plsc.store_scatter(bz, [kvec, mlv], d0z, mask=fm)
                    return carry2

                lax.fori_loop(0, CHUNKQ, q_body, 0)
                col = cidx * CHUNKQ
                pltpu.sync_copy(bx, dpx_h.at[f, :, pl.ds(col, CHUNKQ)])
                pltpu.sync_copy(by, dpy_h.at[f, :, pl.ds(col, CHUNKQ)])
                pltpu.sync_copy(bz, dpz_h.at[f, :, pl.ds(col, CHUNKQ)])
                return carry

            lax.fori_loop(0, NCHUNK, chunk_body, 0)

    return sc_kernel


def _tc_mlp_body(dx_ref, dy_ref, dz_ref, wd_ref, w1_ref, out_ref):
    wd = wd_ref[...]
    w1 = w1_ref[...]
    dxa = dx_ref[0]
    dya = dy_ref[0]
    dza = dz_ref[0]
    dn = (((1,), (0,)), ((), ()))
    acc = jnp.zeros((64, MPT), _f32)
    for k in range(KNB):
        dk = jnp.concatenate(
            [dxa[k:k + 1], dya[k:k + 1], dza[k:k + 1]], axis=0)
        h1 = jnp.maximum(
            lax.dot_general(wd, dk, dn, precision=lax.Precision.HIGHEST,
                            preferred_element_type=_f32), 0.0)
        h2 = jnp.maximum(
            lax.dot_general(w1, h1, dn, precision=lax.Precision.HIGHEST,
                            preferred_element_type=_f32), 0.0)
        acc = jnp.maximum(acc, h2)
    out_ref[0] = acc


def _build_tc_mlp():
    return pl.pallas_call(
        _tc_mlp_body,
        grid=(NFRAME,),
        in_specs=[
            pl.BlockSpec((1, KNB, MPT), lambda f: (f, 0, 0)),
            pl.BlockSpec((1, KNB, MPT), lambda f: (f, 0, 0)),
            pl.BlockSpec((1, KNB, MPT), lambda f: (f, 0, 0)),
            pl.BlockSpec((32, 3), lambda f: (0, 0)),
            pl.BlockSpec((64, 32), lambda f: (0, 0)),
        ],
        out_specs=pl.BlockSpec((1, 64, MPT), lambda f: (f, 0, 0)),
        out_shape=jax.ShapeDtypeStruct((NFRAME, 64, MPT), _f32),
    )


_SC_KERNEL = _build_sc_kernel()
_TC_MLP = _build_tc_mlp()


@jax.jit
def kernel(xyzs, W_d, W1):
    B, T, N, _ = xyzs.shape
    pts = xyzs.reshape(B * T, N, 3)
    xs = pts[:, :, 0]
    ys = pts[:, :, 1]
    zs = pts[:, :, 2]
    rfx, rfy, rfz, dpx, dpy, dpz = _SC_KERNEL(xs, ys, zs)
    feat = _TC_MLP(dpx, dpy, dpz, W_d[:, :3], W1)
    new_xyzs = jnp.stack([rfx, rfy, rfz], axis=-1).reshape(B, T, N // 4, 3)
    new_feats = feat.reshape(B, T, 64, N // 4)
    return new_xyzs, new_feats


# speedup vs baseline: 1.4551x; 1.4551x over previous
import functools

import jax
import jax.numpy as jnp
from jax import lax
from jax.experimental import pallas as pl
from jax.experimental.pallas import tpu as pltpu
from jax.experimental.pallas import tpu_sc as plsc

R2 = 0.81
KNB = 32
NPT = 4096
MPT = NPT // 4
NFRAME = 12
CHUNKQ = 128
NCHUNK = MPT // CHUNKQ
LANES = 16
FPS_UNROLL = 4
BQ_UNROLL = 4

_f32 = jnp.float32
_i32 = jnp.int32


def _build_sc_kernel():
    info = plsc.get_sparse_core_info()
    nc = info.num_cores
    mesh = plsc.VectorSubcoreMesh(core_axis_name="c", subcore_axis_name="s")
    out_type = (
        jax.ShapeDtypeStruct((NFRAME, MPT), _f32),
        jax.ShapeDtypeStruct((NFRAME, MPT), _f32),
        jax.ShapeDtypeStruct((NFRAME, MPT), _f32),
        jax.ShapeDtypeStruct((NFRAME, KNB, MPT), _f32),
        jax.ShapeDtypeStruct((NFRAME, KNB, MPT), _f32),
        jax.ShapeDtypeStruct((NFRAME, KNB, MPT), _f32),
    )
    scratch = [
        pltpu.VMEM((NPT,), _f32),
        pltpu.VMEM((NPT,), _f32),
        pltpu.VMEM((NPT,), _f32),
        pltpu.VMEM((NPT,), _f32),
        pltpu.VMEM((MPT,), _f32),
        pltpu.VMEM((MPT,), _f32),
        pltpu.VMEM((MPT,), _f32),
        pltpu.VMEM((KNB, CHUNKQ), _f32),
        pltpu.VMEM((KNB, CHUNKQ), _f32),
        pltpu.VMEM((KNB, CHUNKQ), _f32),
    ]

    @functools.partial(
        pl.kernel, out_type=out_type, mesh=mesh, scratch_types=scratch,
        compiler_params=pltpu.CompilerParams(needs_layout_passes=False))
    def sc_kernel(xs_h, ys_h, zs_h, rfx_h, rfy_h, rfz_h, dpx_h, dpy_h, dpz_h,
                  xs, ys, zs, dist, rfx, rfy, rfz, bx, by, bz):
        wid = lax.axis_index("s") * nc + lax.axis_index("c")

        @pl.when(wid < NFRAME)
        def _frame():
            f = wid
            pltpu.sync_copy(xs_h.at[f], xs)
            pltpu.sync_copy(ys_h.at[f], ys)
            pltpu.sync_copy(zs_h.at[f], zs)

            iota = lax.iota(_i32, LANES)
            lane0 = iota == 0
            big = jnp.full((LANES,), 1e10, _f32)

            @plsc.parallel_loop(0, NPT // LANES, unroll=8)
            def _init(j):
                dist[pl.ds(j * LANES, LANES)] = big

            def fps_step(i, far):
                farv = jnp.full((LANES,), far, _i32)
                cxv = plsc.load_gather(xs, [farv])
                cyv = plsc.load_gather(ys, [farv])
                czv = plsc.load_gather(zs, [farv])
                iv = jnp.full((LANES,), i, _i32)
                plsc.store_scatter(rfx, [iv], cxv, mask=lane0)
                plsc.store_scatter(rfy, [iv], cyv, mask=lane0)
                plsc.store_scatter(rfz, [iv], czv, mask=lane0)

                neg = jnp.full((LANES,), -1.0, _f32)
                zi = jnp.zeros((LANES,), _i32)

                @plsc.parallel_loop(0, NPT // (LANES * FPS_UNROLL), unroll=4,
                                    carry=((neg,) * FPS_UNROLL,
                                           (zi,) * FPS_UNROLL))
                def inner(j, acc):
                    bv, bi = list(acc[0]), list(acc[1])
                    base = j * (LANES * FPS_UNROLL)
                    for u in range(FPS_UNROLL):
                        off = base + u * LANES
                        dx = xs[pl.ds(off, LANES)] - cxv
                        dy = ys[pl.ds(off, LANES)] - cyv
                        dz = zs[pl.ds(off, LANES)] - czv
                        d = dx * dx + dy * dy + dz * dz
                        nd = jnp.minimum(dist[pl.ds(off, LANES)], d)
                        dist[pl.ds(off, LANES)] = nd
                        gt = nd > bv[u]
                        bv[u] = jnp.where(gt, nd, bv[u])
                        bi[u] = jnp.where(gt, iota + off, bi[u])
                    return (tuple(bv), tuple(bi))

                bv, bi = inner

                def comb(va, ia, vb, ib):
                    t = (vb > va) | ((vb == va) & (ib < ia))
                    return jnp.where(t, vb, va), jnp.where(t, ib, ia)

                v01, i01 = comb(bv[0], bi[0], bv[1], bi[1])
                v23, i23 = comb(bv[2], bi[2], bv[3], bi[3])
                v, ivec = comb(v01, i01, v23, i23)
                mx = jnp.max(v)
                cand = jnp.where(v == mx, ivec, jnp.full((LANES,), 1 << 30, _i32))
                return jnp.min(cand)

            lax.fori_loop(0, MPT, fps_step, jnp.int32(0))

            pltpu.sync_copy(rfx, rfx_h.at[f])
            pltpu.sync_copy(rfy, rfy_h.at[f])
            pltpu.sync_copy(rfz, rfz_h.at[f])

            ones = jnp.ones((LANES,), _i32)

            def chunk_body(cidx, carry):
                def q_body(ml, carry2):
                    m = cidx * CHUNKQ + ml
                    mv = jnp.full((LANES,), m, _i32)
                    qxv = plsc.load_gather(rfx, [mv])
                    qyv = plsc.load_gather(rfy, [mv])
                    qzv = plsc.load_gather(rfz, [mv])
                    mlv = jnp.full((LANES,), ml, _i32)

                    def cond(st):
                        j, cnt = st
                        return (cnt < KNB) & (j < NPT // (LANES * BQ_UNROLL))

                    def body(st):
                        j, cnt = st
                        base = j * (LANES * BQ_UNROLL)
                        dvals = []
                        msks = []
                        for u in range(BQ_UNROLL):
                            off = base + u * LANES
                            dx = xs[pl.ds(off, LANES)] - qxv
                            dy = ys[pl.ds(off, LANES)] - qyv
                            dz = zs[pl.ds(off, LANES)] - qzv
                            d2 = dx * dx + dy * dy + dz * dz
                            dvals.append((dx, dy, dz))
                            msks.append(d2 < R2)
                        anyhit = jnp.any(msks[0] | msks[1] | msks[2] | msks[3])

                        def on_hit():
                            cntv = jnp.full((LANES,), cnt, _i32)
                            for u in range(BQ_UNROLL):
                                dx, dy, dz = dvals[u]
                                msk = msks[u]
                                pos = cntv + plsc.cumsum(ones, mask=msk) - 1
                                smask = msk & (pos < KNB)
                                plsc.store_scatter(bx, [pos, mlv], dx,
                                                   mask=smask)
                                plsc.store_scatter(by, [pos, mlv], dy,
                                                   mask=smask)
                                plsc.store_scatter(bz, [pos, mlv], dz,
                                                   mask=smask)
                                cntv = cntv + plsc.all_reduce_population_count(
                                    msk)
                            return cntv[0]

                        return (j + 1, lax.cond(anyhit, on_hit, lambda: cnt))

                    _, cnt = lax.while_loop(cond, body,
                                            (jnp.int32(0), jnp.int32(0)))

                    cntv = jnp.full((LANES,), cnt, _i32)
                    zi = jnp.zeros((LANES,), _i32)
                    d0x = plsc.load_gather(bx, [zi, mlv])
                    d0y = plsc.load_gather(by, [zi, mlv])
                    d0z = plsc.load_gather(bz, [zi, mlv])
                    for kk in range(KNB // LANES):
                        kvec = iota + kk * LANES
                        fm = kvec >= cntv
                        plsc.store_scatter(bx, [kvec, mlv], d0x, mask=fm)
                        plsc.store_scatter(by, [kvec, mlv], d0y, mask=fm)
                        plsc.store_scatter(bz, [kvec, mlv], d0z, mask=fm)
                    return carry2

                lax.fori_loop(0, CHUNKQ, q_body, 0)
                col = cidx * CHUNKQ
                pltpu.sync_copy(bx, dpx_h.at[f, :, pl.ds(col, CHUNKQ)])
                pltpu.sync_copy(by, dpy_h.at[f, :, pl.ds(col, CHUNKQ)])
                pltpu.sync_copy(bz, dpz_h.at[f, :, pl.ds(col, CHUNKQ)])
                return carry

            lax.fori_loop(0, NCHUNK, chunk_body, 0)

    return sc_kernel


def _tc_mlp_body(dx_ref, dy_ref, dz_ref, wd_ref, w1_ref, out_ref):
    wd = wd_ref[...]
    w1 = w1_ref[...]
    dxa = dx_ref[0]
    dya = dy_ref[0]
    dza = dz_ref[0]
    dn = (((1,), (0,)), ((), ()))
    acc = jnp.zeros((64, MPT), _f32)
    for k in range(KNB):
        dk = jnp.concatenate(
            [dxa[k:k + 1], dya[k:k + 1], dza[k:k + 1]], axis=0)
        h1 = jnp.maximum(
            lax.dot_general(wd, dk, dn, precision=lax.Precision.HIGHEST,
                            preferred_element_type=_f32), 0.0)
        h2 = jnp.maximum(
            lax.dot_general(w1, h1, dn, precision=lax.Precision.HIGHEST,
                            preferred_element_type=_f32), 0.0)
        acc = jnp.maximum(acc, h2)
    out_ref[0] = acc


def _build_tc_mlp():
    return pl.pallas_call(
        _tc_mlp_body,
        grid=(NFRAME,),
        in_specs=[
            pl.BlockSpec((1, KNB, MPT), lambda f: (f, 0, 0)),
            pl.BlockSpec((1, KNB, MPT), lambda f: (f, 0, 0)),
            pl.BlockSpec((1, KNB, MPT), lambda f: (f, 0, 0)),
            pl.BlockSpec((32, 3), lambda f: (0, 0)),
            pl.BlockSpec((64, 32), lambda f: (0, 0)),
        ],
        out_specs=pl.BlockSpec((1, 64, MPT), lambda f: (f, 0, 0)),
        out_shape=jax.ShapeDtypeStruct((NFRAME, 64, MPT), _f32),
    )


_SC_KERNEL = _build_sc_kernel()
_TC_MLP = _build_tc_mlp()


@jax.jit
def kernel(xyzs, W_d, W1):
    B, T, N, _ = xyzs.shape
    pts = xyzs.reshape(B * T, N, 3)
    xs = pts[:, :, 0]
    ys = pts[:, :, 1]
    zs = pts[:, :, 2]
    rfx, rfy, rfz, dpx, dpy, dpz = _SC_KERNEL(xs, ys, zs)
    feat = _TC_MLP(dpx, dpy, dpz, W_d[:, :3], W1)
    new_xyzs = jnp.stack([rfx, rfy, rfz], axis=-1).reshape(B, T, N // 4, 3)
    new_feats = feat.reshape(B, T, 64, N // 4)
    return new_xyzs, new_feats
